# baseline (speedup 1.0000x reference)
import functools

import jax
import jax.numpy as jnp
from jax import lax
from jax.experimental import pallas as pl
from jax.experimental.pallas import tpu as pltpu
from jax.experimental.pallas import tpu_sc as plsc

N_FEAT = 10000
N_OUT = 1024
N_P = 16
N_ROWS = 4096
OUT_B = 8
OUT_H = 128

NC = 2
NS = 16
L = 16
NW = NC * NS
ROWS_PER_W = N_ROWS // NW
R = 4
NGROUPS = ROWS_PER_W // R
NJ = N_OUT // L

_mesh = plsc.VectorSubcoreMesh(core_axis_name="c", subcore_axis_name="s")


@functools.partial(
    pl.kernel,
    out_type=jax.ShapeDtypeStruct((N_ROWS, N_OUT), jnp.float32),
    mesh=_mesh,
    scratch_types=[
        pltpu.VMEM((N_P, N_OUT), jnp.int32),
        pltpu.VMEM((N_P, N_OUT), jnp.float32),
        pltpu.VMEM((2 * R, N_FEAT), jnp.float32),
        pltpu.VMEM((R, N_OUT), jnp.float32),
        pltpu.SemaphoreType.DMA,
        pltpu.SemaphoreType.DMA,
    ],
    compiler_params=pltpu.CompilerParams(
        needs_layout_passes=False, use_tc_tiling_on_sc=True),
)
def _branch_sc(x_hbm, w_hbm, idx_hbm, out_hbm, idx_v, w_v, rows_v, out_v,
               sem0, sem1):
    wid = lax.axis_index("s") * NC + lax.axis_index("c")
    row0 = wid * ROWS_PER_W
    pltpu.sync_copy(idx_hbm, idx_v)
    pltpu.sync_copy(w_hbm, w_v)
    sems = (sem0, sem1)
    pltpu.async_copy(x_hbm.at[pl.ds(row0, R)], rows_v.at[pl.ds(0, R)], sem0)

    @pl.loop(0, NGROUPS, step=2)
    def _group(g0):
        for b in range(2):
            g = g0 + b
            gbase = row0 + g * R

            @pl.when(g + 1 < NGROUPS)
            def _prefetch():
                pltpu.async_copy(
                    x_hbm.at[pl.ds(gbase + R, R)],
                    rows_v.at[pl.ds((1 - b) * R, R)], sems[1 - b])

            pltpu.make_async_copy(
                x_hbm.at[pl.ds(gbase, R)], rows_v.at[pl.ds(b * R, R)],
                sems[b]).wait()

            row_ids = [jnp.full((L,), b * R + r, jnp.int32) for r in range(R)]

            @pl.loop(0, NJ)
            def _j(j):
                col = j * L
                accs = [jnp.zeros((L,), jnp.float32) for _ in range(R)]
                for p in range(N_P):
                    iv = idx_v[p, pl.ds(col, L)]
                    wv = w_v[p, pl.ds(col, L)]
                    for r in range(R):
                        g16 = plsc.load_gather(rows_v, [row_ids[r], iv])
                        accs[r] = accs[r] + g16 * wv
                for r in range(R):
                    out_v[r, pl.ds(col, L)] = accs[r]

            pltpu.sync_copy(out_v, out_hbm.at[pl.ds(gbase, R)])


BM = 256
BKB = 2000


def _tc_body(idx_ref, w_ref, xt_ref, out_ref, m_scr):
    m = pl.program_id(0)

    @pl.when(m == 0)
    def _build():
        for k in range(N_FEAT // BKB):
            base = k * BKB
            riota = (lax.broadcasted_iota(jnp.int16, (BKB, N_OUT), 0)
                     + jnp.int16(base))
            acc = jnp.zeros((BKB, N_OUT), jnp.bfloat16)
            zero = jnp.zeros((BKB, N_OUT), jnp.bfloat16)
            for p in range(N_P):
                ip = idx_ref[pl.ds(p, 1), :].astype(jnp.int16)
                wp = w_ref[pl.ds(p, 1), :].astype(jnp.bfloat16)
                acc = acc + jnp.where(riota == ip, wp, zero)
            m_scr[pl.ds(base, BKB), :] = acc

    xb = xt_ref[...].astype(jnp.bfloat16)
    out_ref[...] = lax.dot_general(xb, m_scr[...],
                                   (((0,), (0,)), ((), ())),
                                   preferred_element_type=jnp.float32)


_branch_tc = pl.pallas_call(
    _tc_body,
    grid=(N_ROWS // BM,),
    in_specs=[
        pl.BlockSpec((N_P, N_OUT), lambda m: (0, 0)),
        pl.BlockSpec((N_P, N_OUT), lambda m: (0, 0)),
        pl.BlockSpec((N_FEAT, BM), lambda m: (0, m)),
    ],
    out_specs=pl.BlockSpec((BM, N_OUT), lambda m: (m, 0)),
    out_shape=jax.ShapeDtypeStruct((N_ROWS, N_OUT), jnp.float32),
    scratch_shapes=[pltpu.VMEM((N_FEAT, N_OUT), jnp.bfloat16)],
    compiler_params=pltpu.CompilerParams(
        dimension_semantics=("arbitrary",)),
)


def kernel(x, weights, all_branch_indices):
    out = _branch_tc(all_branch_indices, weights, x.T)
    return out.reshape(N_ROWS, OUT_B, OUT_H)


# speedup vs baseline: 1.1945x; 1.1945x over previous
import functools

import jax
import jax.numpy as jnp
from jax import lax
from jax.experimental import pallas as pl
from jax.experimental.pallas import tpu as pltpu
from jax.experimental.pallas import tpu_sc as plsc

N_FEAT = 10000
N_OUT = 1024
N_P = 16
N_ROWS = 4096
OUT_B = 8
OUT_H = 128

NC = 2
NS = 16
L = 16
NW = NC * NS
ROWS_PER_W = N_ROWS // NW
R = 4
NGROUPS = ROWS_PER_W // R
NJ = N_OUT // L

_mesh = plsc.VectorSubcoreMesh(core_axis_name="c", subcore_axis_name="s")


@functools.partial(
    pl.kernel,
    out_type=jax.ShapeDtypeStruct((N_ROWS, N_OUT), jnp.float32),
    mesh=_mesh,
    scratch_types=[
        pltpu.VMEM((N_P, N_OUT), jnp.int32),
        pltpu.VMEM((N_P, N_OUT), jnp.float32),
        pltpu.VMEM((2 * R, N_FEAT), jnp.float32),
        pltpu.VMEM((R, N_OUT), jnp.float32),
        pltpu.SemaphoreType.DMA,
        pltpu.SemaphoreType.DMA,
    ],
    compiler_params=pltpu.CompilerParams(
        needs_layout_passes=False, use_tc_tiling_on_sc=True),
)
def _branch_sc(x_hbm, w_hbm, idx_hbm, out_hbm, idx_v, w_v, rows_v, out_v,
               sem0, sem1):
    wid = lax.axis_index("s") * NC + lax.axis_index("c")
    row0 = wid * ROWS_PER_W
    pltpu.sync_copy(idx_hbm, idx_v)
    pltpu.sync_copy(w_hbm, w_v)
    sems = (sem0, sem1)
    pltpu.async_copy(x_hbm.at[pl.ds(row0, R)], rows_v.at[pl.ds(0, R)], sem0)

    @pl.loop(0, NGROUPS, step=2)
    def _group(g0):
        for b in range(2):
            g = g0 + b
            gbase = row0 + g * R

            @pl.when(g + 1 < NGROUPS)
            def _prefetch():
                pltpu.async_copy(
                    x_hbm.at[pl.ds(gbase + R, R)],
                    rows_v.at[pl.ds((1 - b) * R, R)], sems[1 - b])

            pltpu.make_async_copy(
                x_hbm.at[pl.ds(gbase, R)], rows_v.at[pl.ds(b * R, R)],
                sems[b]).wait()

            row_ids = [jnp.full((L,), b * R + r, jnp.int32) for r in range(R)]

            @pl.loop(0, NJ)
            def _j(j):
                col = j * L
                accs = [jnp.zeros((L,), jnp.float32) for _ in range(R)]
                for p in range(N_P):
                    iv = idx_v[p, pl.ds(col, L)]
                    wv = w_v[p, pl.ds(col, L)]
                    for r in range(R):
                        g16 = plsc.load_gather(rows_v, [row_ids[r], iv])
                        accs[r] = accs[r] + g16 * wv
                for r in range(R):
                    out_v[r, pl.ds(col, L)] = accs[r]

            pltpu.sync_copy(out_v, out_hbm.at[pl.ds(gbase, R)])


BM = 512
BK = 5000
KBLOCKS = N_FEAT // BK
MBLOCKS = N_ROWS // BM


def _tc_body(idx_ref, w_ref, xt_ref, out_ref, m_scr):
    m = pl.program_id(0)
    k = pl.program_id(1)
    base = k * BK

    @pl.when(m == 0)
    def _build():
        base16 = lax.convert_element_type(base, jnp.int16)
        riota = lax.broadcasted_iota(jnp.int16, (BK, N_OUT), 0) + base16
        acc = jnp.zeros((BK, N_OUT), jnp.bfloat16)
        zero = jnp.zeros((BK, N_OUT), jnp.bfloat16)
        for p in range(N_P):
            ip = idx_ref[pl.ds(p, 1), :].astype(jnp.int16)
            wp = w_ref[pl.ds(p, 1), :].astype(jnp.bfloat16)
            acc = acc + jnp.where(riota == ip, wp, zero)
        m_scr[pl.ds(base, BK), :] = acc

    xb = xt_ref[...].astype(jnp.bfloat16)
    prod = lax.dot_general(xb, m_scr[pl.ds(base, BK), :],
                           (((0,), (0,)), ((), ())),
                           preferred_element_type=jnp.float32)

    @pl.when(k == 0)
    def _init():
        out_ref[...] = prod

    @pl.when(k != 0)
    def _accum():
        out_ref[...] = out_ref[...] + prod


_branch_tc = pl.pallas_call(
    _tc_body,
    grid=(MBLOCKS, KBLOCKS),
    in_specs=[
        pl.BlockSpec((N_P, N_OUT), lambda m, k: (0, 0)),
        pl.BlockSpec((N_P, N_OUT), lambda m, k: (0, 0)),
        pl.BlockSpec((BK, BM), lambda m, k: (k, m)),
    ],
    out_specs=pl.BlockSpec((BM, N_OUT), lambda m, k: (m, 0)),
    out_shape=jax.ShapeDtypeStruct((N_ROWS, N_OUT), jnp.float32),
    scratch_shapes=[pltpu.VMEM((N_FEAT, N_OUT), jnp.bfloat16)],
    compiler_params=pltpu.CompilerParams(
        dimension_semantics=("arbitrary", "arbitrary")),
)


def kernel(x, weights, all_branch_indices):
    out = _branch_tc(all_branch_indices, weights, x.T)
    return out.reshape(N_ROWS, OUT_B, OUT_H)


# speedup vs baseline: 1.1987x; 1.0035x over previous
import jax
import jax.numpy as jnp
from jax import lax
from jax.experimental import pallas as pl
from jax.experimental.pallas import tpu as pltpu

N_FEAT = 10000
N_OUT = 1024
N_P = 16
N_ROWS = 4096
OUT_B = 8
OUT_H = 128

BM = 512
BK = 5000
KBLOCKS = N_FEAT // BK
MBLOCKS = N_ROWS // BM


def _tc_body(idx_ref, w_ref, xt_ref, out_ref, m_scr):
    m = pl.program_id(0)
    k = pl.program_id(1)
    base = k * BK

    @pl.when(m == 0)
    def _build():
        base16 = lax.convert_element_type(base, jnp.int16)
        riota = lax.broadcasted_iota(jnp.int16, (BK, N_OUT), 0) + base16
        acc = jnp.zeros((BK, N_OUT), jnp.bfloat16)
        zero = jnp.zeros((BK, N_OUT), jnp.bfloat16)
        for p in range(N_P):
            ip = idx_ref[pl.ds(p, 1), :].astype(jnp.int16)
            wp = w_ref[pl.ds(p, 1), :].astype(jnp.bfloat16)
            acc = acc + jnp.where(riota == ip, wp, zero)
        m_scr[pl.ds(base, BK), :] = acc

    xb = xt_ref[...].astype(jnp.bfloat16)
    prod = lax.dot_general(xb, m_scr[pl.ds(base, BK), :],
                           (((0,), (0,)), ((), ())),
                           preferred_element_type=jnp.float32)

    @pl.when(k == 0)
    def _init():
        out_ref[...] = prod

    @pl.when(k != 0)
    def _accum():
        out_ref[...] = out_ref[...] + prod


_branch_tc = pl.pallas_call(
    _tc_body,
    grid=(MBLOCKS, KBLOCKS),
    in_specs=[
        pl.BlockSpec((N_P, N_OUT), lambda m, k: (0, 0)),
        pl.BlockSpec((N_P, N_OUT), lambda m, k: (0, 0)),
        pl.BlockSpec((BK, BM), lambda m, k: (k, m)),
    ],
    out_specs=pl.BlockSpec((BM, N_OUT), lambda m, k: (m, 0)),
    out_shape=jax.ShapeDtypeStruct((N_ROWS, N_OUT), jnp.float32),
    scratch_shapes=[pltpu.VMEM((N_FEAT, N_OUT), jnp.bfloat16)],
    compiler_params=pltpu.CompilerParams(
        dimension_semantics=("arbitrary", "arbitrary")),
)


def kernel(x, weights, all_branch_indices):
    out = _branch_tc(all_branch_indices, weights, x.T)
    return out.reshape(N_ROWS, OUT_B, OUT_H)
